# baseline (speedup 1.0000x reference)
import functools

import jax
import jax.numpy as jnp
from jax import lax
from jax.experimental import pallas as pl
from jax.experimental.pallas import tpu as pltpu
from jax.experimental.pallas import tpu_sc as plsc

E = 8
K = 2
D = 1024
F = 1408
FS = 5632
N = 2048

TILE_G = 128
P = N * K + E * TILE_G
NT_G = P // TILE_G
PCH = 512
TOK_TILE = 512
FS_CHUNK = 512

SC_NC = 2
SC_NS = 16
SC_NW = SC_NC * SC_NS
SC_CH = 32


def _router_body(x_ref, wr_ref, w_ref, pos_ref, src_ref, meta_ref):
    x = x_ref[...]
    logits = lax.dot_general(x, wr_ref[...], (((1,), (1,)), ((), ())),
                             preferred_element_type=jnp.float32)
    probs = jax.nn.softmax(logits, axis=-1)
    iota8 = lax.broadcasted_iota(jnp.int32, (N, E), 1)
    m1 = jnp.max(probs, axis=-1, keepdims=True)
    a1 = jnp.min(jnp.where(probs == m1, iota8, E), axis=-1, keepdims=True)
    masked = jnp.where(iota8 == a1, -jnp.inf, probs)
    m2 = jnp.max(masked, axis=-1, keepdims=True)
    a2 = jnp.min(jnp.where(masked == m2, iota8, E), axis=-1, keepdims=True)
    w_ref[...] = jnp.concatenate([m1, m2], axis=1)

    oh1 = jnp.where(iota8 == a1, 1.0, 0.0)
    oh2 = jnp.where(iota8 == a2, 1.0, 0.0)
    hist = oh1 + oh2

    r = lax.broadcasted_iota(jnp.int32, (N, N), 0)
    c = lax.broadcasted_iota(jnp.int32, (N, N), 1)
    ltri = jnp.where(r > c, 1.0, 0.0)
    pfx = lax.dot_general(ltri, hist, (((1,), (0,)), ((), ())),
                          preferred_element_type=jnp.float32)

    counts = jnp.sum(hist, axis=0, keepdims=True)
    aligned = jnp.floor((counts + (TILE_G - 1)) / TILE_G) * TILE_G
    e_r = lax.broadcasted_iota(jnp.int32, (E, E), 0)
    e_c = lax.broadcasted_iota(jnp.int32, (E, E), 1)
    excl = jnp.where(e_r < e_c, 1.0, 0.0)
    incl = jnp.where(e_r <= e_c, 1.0, 0.0)
    seg_start = lax.dot_general(aligned, excl, (((1,), (0,)), ((), ())),
                                preferred_element_type=jnp.float32)
    ends = lax.dot_general(aligned, incl, (((1,), (0,)), ((), ())),
                           preferred_element_type=jnp.float32)

    rank1 = jnp.sum(jnp.where(iota8 == a1, pfx, 0.0), axis=1, keepdims=True)
    rank2 = jnp.sum(jnp.where(iota8 == a2, pfx, 0.0), axis=1, keepdims=True)
    base1 = jnp.sum(jnp.where(iota8 == a1, seg_start, 0.0), axis=1,
                    keepdims=True)
    base2 = jnp.sum(jnp.where(iota8 == a2, seg_start, 0.0), axis=1,
                    keepdims=True)
    pos1 = base1 + rank1
    pos2 = base2 + rank2
    pos_ref[...] = jnp.concatenate([pos1, pos2], axis=1).astype(jnp.int32)

    g128 = lax.broadcasted_iota(jnp.int32, (1, 128), 1).astype(jnp.float32)
    gstart = g128 * TILE_G
    te = jnp.zeros((1, 128), jnp.float32)
    for e in range(E):
        te = te + jnp.where(gstart >= ends[:, e:e + 1], 1.0, 0.0)
    te = jnp.minimum(te, E - 1)
    meta_ref[...] = jnp.broadcast_to(te, (8, 128)).astype(jnp.int32)

    tokrow = lax.broadcasted_iota(jnp.int32, (1, N), 1).astype(jnp.float32)
    pcol = lax.broadcasted_iota(jnp.int32, (1, PCH), 1).astype(jnp.float32)
    for ci in range(P // PCH):
        pc = pcol + (ci * PCH)
        match = (jnp.where(pos1 == pc, 1.0, 0.0) +
                 jnp.where(pos2 == pc, 1.0, 0.0))
        src = lax.dot_general(tokrow, match, (((1,), (0,)), ((), ())),
                              preferred_element_type=jnp.float32)
        hit = jnp.sum(match, axis=0, keepdims=True)
        fill = pc - jnp.floor(pc / N) * N
        src = src + (1.0 - hit) * fill
        src_ref[pl.ds(ci, 1), :] = src.astype(jnp.int32)


def _sc_gather(table, idx3d, n_rows):
    d = table.shape[1]
    rows_per_w = n_rows // SC_NW
    n_chunks = idx3d.shape[1]
    ch_rows = idx3d.shape[2]
    mesh = plsc.VectorSubcoreMesh(core_axis_name="c", subcore_axis_name="s")

    @functools.partial(
        pl.kernel,
        out_type=jax.ShapeDtypeStruct((n_rows, d), jnp.float32),
        mesh=mesh,
        cost_estimate=pl.CostEstimate(
            flops=0, transcendentals=0,
            bytes_accessed=2 * n_rows * d * 4),
        scratch_types=[
            pltpu.VMEM((n_chunks, ch_rows), jnp.int32),
            pltpu.VMEM((ch_rows, d), jnp.float32),
            pltpu.VMEM((ch_rows, d), jnp.float32),
            pltpu.SemaphoreType.DMA,
            pltpu.SemaphoreType.DMA,
            pltpu.SemaphoreType.DMA,
            pltpu.SemaphoreType.DMA,
        ],
    )
    def gather_kernel(table_hbm, idx_hbm, out_hbm, idx_v, rows0, rows1,
                      g0, g1, s0, s1):
        rows = (rows0, rows1)
        gsem = (g0, g1)
        ssem = (s0, s1)
        wid = lax.axis_index("s") * SC_NC + lax.axis_index("c")
        row0 = wid * rows_per_w
        pltpu.sync_copy(idx_hbm.at[wid], idx_v)
        gathers = {}
        stores = {}
        gathers[0] = pltpu.async_copy(table_hbm.at[idx_v.at[0]], rows[0],
                                      gsem[0])
        for ch in range(n_chunks):
            b = ch & 1
            gathers[ch].wait()
            if ch + 1 < n_chunks:
                nb = (ch + 1) & 1
                if ch - 1 >= 0:
                    stores[ch - 1].wait()
                gathers[ch + 1] = pltpu.async_copy(
                    table_hbm.at[idx_v.at[ch + 1]], rows[nb], gsem[nb])
            stores[ch] = pltpu.async_copy(
                rows[b], out_hbm.at[pl.ds(row0 + ch * ch_rows, ch_rows)],
                ssem[b])
        for ch in (n_chunks - 2, n_chunks - 1):
            if ch >= 0:
                stores[ch].wait()

    return gather_kernel(table, idx3d)


def _grouped_body(te_ref, disp_ref, wgu_ref, wd_ref, out_ref):
    merged = jnp.dot(disp_ref[...], wgu_ref[0],
                     preferred_element_type=jnp.float32)
    h = jax.nn.silu(merged[:, :F]) * merged[:, F:]
    out_ref[...] = jnp.dot(h, wd_ref[0], preferred_element_type=jnp.float32)


def _shared1_body(x_ref, wg_ref, wu_ref, wd_ref, out_ref, acc_ref):
    j = pl.program_id(1)
    nj = pl.num_programs(1)
    x = x_ref[...]
    g = lax.dot_general(x, wg_ref[...], (((1,), (1,)), ((), ())),
                        preferred_element_type=jnp.float32)
    u = lax.dot_general(x, wu_ref[...], (((1,), (1,)), ((), ())),
                        preferred_element_type=jnp.float32)
    h = jax.nn.silu(g) * u
    o = lax.dot_general(h, wd_ref[...], (((1,), (1,)), ((), ())),
                        preferred_element_type=jnp.float32)

    @pl.when(j == 0)
    def _():
        acc_ref[...] = jnp.zeros_like(acc_ref)

    acc_ref[...] += o

    @pl.when(j == nj - 1)
    def _():
        out_ref[...] = acc_ref[...]


def _shared2_body(x_ref, p1_ref, wg_ref, wu_ref, wd_ref, gw_ref, out_ref,
                  acc_ref):
    j = pl.program_id(1)
    nj = pl.num_programs(1)
    x = x_ref[...]
    g = lax.dot_general(x, wg_ref[...], (((1,), (1,)), ((), ())),
                        preferred_element_type=jnp.float32)
    u = lax.dot_general(x, wu_ref[...], (((1,), (1,)), ((), ())),
                        preferred_element_type=jnp.float32)
    h = jax.nn.silu(g) * u
    o = lax.dot_general(h, wd_ref[...], (((1,), (1,)), ((), ())),
                        preferred_element_type=jnp.float32)

    @pl.when(j == 0)
    def _():
        acc_ref[...] = p1_ref[...]

    acc_ref[...] += o

    @pl.when(j == nj - 1)
    def _():
        sg = jax.nn.sigmoid(
            lax.dot_general(x, gw_ref[...], (((1,), (1,)), ((), ())),
                            preferred_element_type=jnp.float32))
        out_ref[...] = sg * acc_ref[...]


def _combine_body(u0_ref, u1_ref, w0_ref, w1_ref, sh_ref, out_ref):
    out_ref[...] = (w0_ref[...] * u0_ref[0] +
                    w1_ref[...] * u1_ref[0] + sh_ref[...])


@jax.jit
def kernel(hidden_states, router_weight, merged_gate_up_proj,
           merged_down_proj, shared_gate_up_w, shared_down_w, shared_gate_w):
    x = hidden_states

    w, pos, src_tok, meta = pl.pallas_call(
        _router_body,
        out_shape=[
            jax.ShapeDtypeStruct((N, K), jnp.float32),
            jax.ShapeDtypeStruct((N, K), jnp.int32),
            jax.ShapeDtypeStruct((P // PCH, PCH), jnp.int32),
            jax.ShapeDtypeStruct((8, 128), jnp.int32),
        ],
    )(x, router_weight)

    nt = N // TOK_TILE
    nj = FS // FS_CHUNK
    nj1 = 2
    nj2 = nj - nj1
    shared1 = pl.pallas_call(
        _shared1_body,
        grid=(nt, nj1),
        in_specs=[
            pl.BlockSpec((TOK_TILE, D), lambda t, j: (t, 0)),
            pl.BlockSpec((FS_CHUNK, D), lambda t, j: (j, 0)),
            pl.BlockSpec((FS_CHUNK, D), lambda t, j: (j + FS // FS_CHUNK, 0)),
            pl.BlockSpec((D, FS_CHUNK), lambda t, j: (0, j)),
        ],
        out_specs=pl.BlockSpec((TOK_TILE, D), lambda t, j: (t, 0)),
        out_shape=jax.ShapeDtypeStruct((N, D), jnp.float32),
        scratch_shapes=[pltpu.VMEM((TOK_TILE, D), jnp.float32)],
        compiler_params=pltpu.CompilerParams(
            dimension_semantics=("parallel", "arbitrary")),
    )(x, shared_gate_up_w, shared_gate_up_w, shared_down_w)

    disp = _sc_gather(x, src_tok.reshape(SC_NW, 4, P // SC_NW // 4), P)

    te_arr = meta[0, :NT_G] + (shared1[0, 0] * 0.0).astype(jnp.int32)
    grid_spec = pltpu.PrefetchScalarGridSpec(
        num_scalar_prefetch=1,
        grid=(NT_G,),
        in_specs=[
            pl.BlockSpec((TILE_G, D), lambda g, te: (g, 0)),
            pl.BlockSpec((1, D, 2 * F), lambda g, te: (te[g], 0, 0)),
            pl.BlockSpec((1, F, D), lambda g, te: (te[g], 0, 0)),
        ],
        out_specs=pl.BlockSpec((TILE_G, D), lambda g, te: (g, 0)),
    )
    eo = pl.pallas_call(
        _grouped_body,
        grid_spec=grid_spec,
        out_shape=jax.ShapeDtypeStruct((P, D), jnp.float32),
        compiler_params=pltpu.CompilerParams(
            dimension_semantics=("arbitrary",)),
    )(te_arr, disp, merged_gate_up_proj, merged_down_proj)

    unperm = _sc_gather(eo, pos.T.reshape(SC_NW, 4, N * K // SC_NW // 4),
                        N * K)
    shared = pl.pallas_call(
        _shared2_body,
        grid=(nt, nj2),
        in_specs=[
            pl.BlockSpec((TOK_TILE, D), lambda t, j: (t, 0)),
            pl.BlockSpec((TOK_TILE, D), lambda t, j: (t, 0)),
            pl.BlockSpec((FS_CHUNK, D), lambda t, j: (j + 2, 0)),
            pl.BlockSpec((FS_CHUNK, D),
                         lambda t, j: (j + 2 + FS // FS_CHUNK, 0)),
            pl.BlockSpec((D, FS_CHUNK), lambda t, j: (0, j + 2)),
            pl.BlockSpec((1, D), lambda t, j: (0, 0)),
        ],
        out_specs=pl.BlockSpec((TOK_TILE, D), lambda t, j: (t, 0)),
        out_shape=jax.ShapeDtypeStruct((N, D), jnp.float32),
        scratch_shapes=[pltpu.VMEM((TOK_TILE, D), jnp.float32)],
        compiler_params=pltpu.CompilerParams(
            dimension_semantics=("parallel", "arbitrary")),
    )(x, shared1, shared_gate_up_w, shared_gate_up_w, shared_down_w,
      shared_gate_w)

    u3 = unperm.reshape(K, N, D)
    w0 = w[:, 0:1]
    w1 = w[:, 1:2]

    out = pl.pallas_call(
        _combine_body,
        grid=(nt,),
        in_specs=[
            pl.BlockSpec((1, TOK_TILE, D), lambda t: (0, t, 0)),
            pl.BlockSpec((1, TOK_TILE, D), lambda t: (1, t, 0)),
            pl.BlockSpec((TOK_TILE, 1), lambda t: (t, 0)),
            pl.BlockSpec((TOK_TILE, 1), lambda t: (t, 0)),
            pl.BlockSpec((TOK_TILE, D), lambda t: (t, 0)),
        ],
        out_specs=pl.BlockSpec((TOK_TILE, D), lambda t: (t, 0)),
        out_shape=jax.ShapeDtypeStruct((N, D), jnp.float32),
        compiler_params=pltpu.CompilerParams(
            dimension_semantics=("parallel",)),
    )(u3, u3, w0, w1, shared)

    return out


# speedup vs baseline: 1.0226x; 1.0226x over previous
import functools

import jax
import jax.numpy as jnp
from jax import lax
from jax.experimental import pallas as pl
from jax.experimental.pallas import tpu as pltpu
from jax.experimental.pallas import tpu_sc as plsc

E = 8
K = 2
D = 1024
F = 1408
FS = 5632
N = 2048

TILE_G = 128
P = N * K + E * TILE_G
NT_G = P // TILE_G
PCH = 512
TOK_TILE = 512
FS_CHUNK = 512

SC_NC = 2
SC_NS = 16
SC_NW = SC_NC * SC_NS
SC_CH = 32


def _router_body(x_ref, wr_ref, w_ref, pos_ref, src_ref, meta_ref):
    x = x_ref[...]
    logits = lax.dot_general(x, wr_ref[...], (((1,), (1,)), ((), ())),
                             preferred_element_type=jnp.float32)
    probs = jax.nn.softmax(logits, axis=-1)
    iota8 = lax.broadcasted_iota(jnp.int32, (N, E), 1)
    m1 = jnp.max(probs, axis=-1, keepdims=True)
    a1 = jnp.min(jnp.where(probs == m1, iota8, E), axis=-1, keepdims=True)
    masked = jnp.where(iota8 == a1, -jnp.inf, probs)
    m2 = jnp.max(masked, axis=-1, keepdims=True)
    a2 = jnp.min(jnp.where(masked == m2, iota8, E), axis=-1, keepdims=True)
    w_ref[...] = jnp.concatenate([m1, m2], axis=1)

    oh1 = jnp.where(iota8 == a1, 1.0, 0.0)
    oh2 = jnp.where(iota8 == a2, 1.0, 0.0)
    hist = oh1 + oh2

    r = lax.broadcasted_iota(jnp.int32, (N, N), 0)
    c = lax.broadcasted_iota(jnp.int32, (N, N), 1)
    ltri = jnp.where(r > c, 1.0, 0.0)
    pfx = lax.dot_general(ltri, hist, (((1,), (0,)), ((), ())),
                          preferred_element_type=jnp.float32)

    counts = jnp.sum(hist, axis=0, keepdims=True)
    aligned = jnp.floor((counts + (TILE_G - 1)) / TILE_G) * TILE_G
    e_r = lax.broadcasted_iota(jnp.int32, (E, E), 0)
    e_c = lax.broadcasted_iota(jnp.int32, (E, E), 1)
    excl = jnp.where(e_r < e_c, 1.0, 0.0)
    incl = jnp.where(e_r <= e_c, 1.0, 0.0)
    seg_start = lax.dot_general(aligned, excl, (((1,), (0,)), ((), ())),
                                preferred_element_type=jnp.float32)
    ends = lax.dot_general(aligned, incl, (((1,), (0,)), ((), ())),
                           preferred_element_type=jnp.float32)

    rank1 = jnp.sum(jnp.where(iota8 == a1, pfx, 0.0), axis=1, keepdims=True)
    rank2 = jnp.sum(jnp.where(iota8 == a2, pfx, 0.0), axis=1, keepdims=True)
    base1 = jnp.sum(jnp.where(iota8 == a1, seg_start, 0.0), axis=1,
                    keepdims=True)
    base2 = jnp.sum(jnp.where(iota8 == a2, seg_start, 0.0), axis=1,
                    keepdims=True)
    pos1 = base1 + rank1
    pos2 = base2 + rank2
    pos_ref[...] = jnp.concatenate([pos1, pos2], axis=1).astype(jnp.int32)

    g128 = lax.broadcasted_iota(jnp.int32, (1, 128), 1).astype(jnp.float32)
    gstart = g128 * TILE_G
    te = jnp.zeros((1, 128), jnp.float32)
    for e in range(E):
        te = te + jnp.where(gstart >= ends[:, e:e + 1], 1.0, 0.0)
    te = jnp.minimum(te, E - 1)
    meta_ref[...] = jnp.broadcast_to(te, (8, 128)).astype(jnp.int32)

    tokrow = lax.broadcasted_iota(jnp.int32, (1, N), 1).astype(jnp.float32)
    pcol = lax.broadcasted_iota(jnp.int32, (1, PCH), 1).astype(jnp.float32)
    for ci in range(P // PCH):
        pc = pcol + (ci * PCH)
        match = (jnp.where(pos1 == pc, 1.0, 0.0) +
                 jnp.where(pos2 == pc, 1.0, 0.0))
        src = lax.dot_general(tokrow, match, (((1,), (0,)), ((), ())),
                              preferred_element_type=jnp.float32)
        hit = jnp.sum(match, axis=0, keepdims=True)
        fill = pc - jnp.floor(pc / N) * N
        src = src + (1.0 - hit) * fill
        src_ref[pl.ds(ci, 1), :] = src.astype(jnp.int32)


def _sc_gather(table, idx3d, n_rows):
    d = table.shape[1]
    rows_per_w = n_rows // SC_NW
    n_chunks = idx3d.shape[1]
    ch_rows = idx3d.shape[2]
    mesh = plsc.VectorSubcoreMesh(core_axis_name="c", subcore_axis_name="s")

    @functools.partial(
        pl.kernel,
        out_type=jax.ShapeDtypeStruct((n_rows, d), jnp.float32),
        mesh=mesh,
        cost_estimate=pl.CostEstimate(
            flops=0, transcendentals=0,
            bytes_accessed=2 * n_rows * d * 4),
        scratch_types=[
            pltpu.VMEM((n_chunks, ch_rows), jnp.int32),
            pltpu.VMEM((ch_rows, d), jnp.float32),
            pltpu.VMEM((ch_rows, d), jnp.float32),
            pltpu.SemaphoreType.DMA,
            pltpu.SemaphoreType.DMA,
            pltpu.SemaphoreType.DMA,
            pltpu.SemaphoreType.DMA,
        ],
    )
    def gather_kernel(table_hbm, idx_hbm, out_hbm, idx_v, rows0, rows1,
                      g0, g1, s0, s1):
        rows = (rows0, rows1)
        gsem = (g0, g1)
        ssem = (s0, s1)
        wid = lax.axis_index("s") * SC_NC + lax.axis_index("c")
        row0 = wid * rows_per_w
        pltpu.sync_copy(idx_hbm.at[wid], idx_v)
        gathers = {}
        stores = {}
        gathers[0] = pltpu.async_copy(table_hbm.at[idx_v.at[0]], rows[0],
                                      gsem[0])
        for ch in range(n_chunks):
            b = ch & 1
            gathers[ch].wait()
            if ch + 1 < n_chunks:
                nb = (ch + 1) & 1
                if ch - 1 >= 0:
                    stores[ch - 1].wait()
                gathers[ch + 1] = pltpu.async_copy(
                    table_hbm.at[idx_v.at[ch + 1]], rows[nb], gsem[nb])
            stores[ch] = pltpu.async_copy(
                rows[b], out_hbm.at[pl.ds(row0 + ch * ch_rows, ch_rows)],
                ssem[b])
        for ch in (n_chunks - 2, n_chunks - 1):
            if ch >= 0:
                stores[ch].wait()

    return gather_kernel(table, idx3d)


def _grouped_body(te_ref, disp_ref, wgu_ref, wd_ref, out_ref):
    merged = jnp.dot(disp_ref[...], wgu_ref[0],
                     preferred_element_type=jnp.float32)
    h = jax.nn.silu(merged[:, :F]) * merged[:, F:]
    out_ref[...] = jnp.dot(h, wd_ref[0], preferred_element_type=jnp.float32)


def _shared_body(x_ref, wg_ref, wu_ref, wd_ref, gw_ref, out_ref, acc_ref):
    j = pl.program_id(1)
    nj = pl.num_programs(1)
    x = x_ref[...]
    g = lax.dot_general(x, wg_ref[...], (((1,), (1,)), ((), ())),
                        preferred_element_type=jnp.float32)
    u = lax.dot_general(x, wu_ref[...], (((1,), (1,)), ((), ())),
                        preferred_element_type=jnp.float32)
    h = jax.nn.silu(g) * u
    o = lax.dot_general(h, wd_ref[...], (((1,), (1,)), ((), ())),
                        preferred_element_type=jnp.float32)

    @pl.when(j == 0)
    def _():
        acc_ref[...] = jnp.zeros_like(acc_ref)

    acc_ref[...] += o

    @pl.when(j == nj - 1)
    def _():
        sg = jax.nn.sigmoid(
            lax.dot_general(x, gw_ref[...], (((1,), (1,)), ((), ())),
                            preferred_element_type=jnp.float32))
        out_ref[...] = sg * acc_ref[...]


def _combine_body(u0_ref, u1_ref, w0_ref, w1_ref, sh_ref, out_ref):
    out_ref[...] = (w0_ref[...] * u0_ref[0] +
                    w1_ref[...] * u1_ref[0] + sh_ref[...])


@jax.jit
def kernel(hidden_states, router_weight, merged_gate_up_proj,
           merged_down_proj, shared_gate_up_w, shared_down_w, shared_gate_w):
    x = hidden_states

    w, pos, src_tok, meta = pl.pallas_call(
        _router_body,
        out_shape=[
            jax.ShapeDtypeStruct((N, K), jnp.float32),
            jax.ShapeDtypeStruct((N, K), jnp.int32),
            jax.ShapeDtypeStruct((P // PCH, PCH), jnp.int32),
            jax.ShapeDtypeStruct((8, 128), jnp.int32),
        ],
    )(x, router_weight)

    nt = N // TOK_TILE
    nj = FS // FS_CHUNK
    shared = pl.pallas_call(
        _shared_body,
        grid=(nt, nj),
        in_specs=[
            pl.BlockSpec((TOK_TILE, D), lambda t, j: (t, 0)),
            pl.BlockSpec((FS_CHUNK, D), lambda t, j: (j, 0)),
            pl.BlockSpec((FS_CHUNK, D), lambda t, j: (j + FS // FS_CHUNK, 0)),
            pl.BlockSpec((D, FS_CHUNK), lambda t, j: (0, j)),
            pl.BlockSpec((1, D), lambda t, j: (0, 0)),
        ],
        out_specs=pl.BlockSpec((TOK_TILE, D), lambda t, j: (t, 0)),
        out_shape=jax.ShapeDtypeStruct((N, D), jnp.float32),
        scratch_shapes=[pltpu.VMEM((TOK_TILE, D), jnp.float32)],
        compiler_params=pltpu.CompilerParams(
            dimension_semantics=("parallel", "arbitrary")),
    )(x, shared_gate_up_w, shared_gate_up_w, shared_down_w, shared_gate_w)

    disp = _sc_gather(x, src_tok.reshape(SC_NW, 4, P // SC_NW // 4), P)

    te_arr = meta[0, :NT_G]
    grid_spec = pltpu.PrefetchScalarGridSpec(
        num_scalar_prefetch=1,
        grid=(NT_G,),
        in_specs=[
            pl.BlockSpec((TILE_G, D), lambda g, te: (g, 0)),
            pl.BlockSpec((1, D, 2 * F), lambda g, te: (te[g], 0, 0)),
            pl.BlockSpec((1, F, D), lambda g, te: (te[g], 0, 0)),
        ],
        out_specs=pl.BlockSpec((TILE_G, D), lambda g, te: (g, 0)),
    )
    eo = pl.pallas_call(
        _grouped_body,
        grid_spec=grid_spec,
        out_shape=jax.ShapeDtypeStruct((P, D), jnp.float32),
        compiler_params=pltpu.CompilerParams(
            dimension_semantics=("arbitrary",)),
    )(te_arr, disp, merged_gate_up_proj, merged_down_proj)

    unperm = _sc_gather(eo, pos.T.reshape(SC_NW, 4, N * K // SC_NW // 4),
                        N * K)
    u3 = unperm.reshape(K, N, D)
    w0 = w[:, 0:1]
    w1 = w[:, 1:2]

    out = pl.pallas_call(
        _combine_body,
        grid=(nt,),
        in_specs=[
            pl.BlockSpec((1, TOK_TILE, D), lambda t: (0, t, 0)),
            pl.BlockSpec((1, TOK_TILE, D), lambda t: (1, t, 0)),
            pl.BlockSpec((TOK_TILE, 1), lambda t: (t, 0)),
            pl.BlockSpec((TOK_TILE, 1), lambda t: (t, 0)),
            pl.BlockSpec((TOK_TILE, D), lambda t: (t, 0)),
        ],
        out_specs=pl.BlockSpec((TOK_TILE, D), lambda t: (t, 0)),
        out_shape=jax.ShapeDtypeStruct((N, D), jnp.float32),
        compiler_params=pltpu.CompilerParams(
            dimension_semantics=("parallel",)),
    )(u3, u3, w0, w1, shared)

    return out
